# baseline (speedup 1.0000x reference)
import jax
import jax.numpy as jnp
from jax.experimental import pallas as pl
from jax.experimental.pallas import tpu as pltpu

_B, _L, _H = 32, 2048, 1024
_K = 3 * _H
_LB = 256
_NL = _L // _LB


def _decoder_kernel(trg_ref, w_ref, b_ref, norm_ref, summ_ref,
                    m_ref, s_ref, acc_ref):
    j = pl.program_id(1)

    @pl.when(j == 0)
    def _init():
        m_ref[...] = jnp.full_like(m_ref, -jnp.inf)
        s_ref[...] = jnp.zeros_like(s_ref)
        acc_ref[...] = jnp.zeros_like(acc_ref)

    t = jnp.dot(trg_ref[0], w_ref[...],
                preferred_element_type=jnp.float32) + b_ref[...]

    sq = jnp.sum(t * t, axis=1, keepdims=True)
    norm = jnp.sqrt(sq)
    norm_ref[0, 0] = norm

    mb = jnp.max(norm, axis=0, keepdims=True)
    m_old = m_ref[...]
    m_new = jnp.maximum(m_old, mb)
    alpha = jnp.exp(m_old - m_new)
    p = jnp.exp(norm - m_new[0:1, 0:1])
    sb = jnp.sum(p, axis=0, keepdims=True)
    s_ref[...] = alpha * s_ref[...] + sb
    wsum = jnp.sum(t * p, axis=0, keepdims=True)
    acc_ref[...] = alpha[0:1, 0:1] * acc_ref[...] + wsum
    m_ref[...] = m_new

    @pl.when(j == _NL - 1)
    def _finish():
        summ_ref[0] = acc_ref[...] / s_ref[0:1, 0:1]


def kernel(trg, src, fc_w, fc_b):
    del src
    w_t = fc_w.T
    b2 = fc_b.reshape(1, _H)

    norm4, summ3 = pl.pallas_call(
        _decoder_kernel,
        grid=(_B, _NL),
        in_specs=[
            pl.BlockSpec((1, _LB, _K), lambda b, j: (b, j, 0)),
            pl.BlockSpec((_K, _H), lambda b, j: (0, 0)),
            pl.BlockSpec((1, _H), lambda b, j: (0, 0)),
        ],
        out_specs=[
            pl.BlockSpec((1, 1, _LB, 1), lambda b, j: (b, j, 0, 0)),
            pl.BlockSpec((1, 1, _H), lambda b, j: (b, 0, 0)),
        ],
        out_shape=[
            jax.ShapeDtypeStruct((_B, _NL, _LB, 1), jnp.float32),
            jax.ShapeDtypeStruct((_B, 1, _H), jnp.float32),
        ],
        scratch_shapes=[
            pltpu.VMEM((1, 128), jnp.float32),
            pltpu.VMEM((1, 128), jnp.float32),
            pltpu.VMEM((1, _H), jnp.float32),
        ],
        compiler_params=pltpu.CompilerParams(
            dimension_semantics=("parallel", "arbitrary"),
            vmem_limit_bytes=56 * 1024 * 1024,
        ),
    )(trg, w_t, b2)

    return summ3.reshape(_B, _H), norm4.reshape(_B, _L)


# speedup vs baseline: 1.1865x; 1.1865x over previous
import jax
import jax.numpy as jnp
from jax.experimental import pallas as pl
from jax.experimental.pallas import tpu as pltpu

_B, _L, _H = 32, 2048, 1024
_K = 3 * _H
_CH = 256
_NCH = 4
_LB = _CH * _NCH
_NL = _L // _LB


def _decoder_kernel(trg_ref, w_ref, b_ref, norm_ref, summ_ref,
                    m_ref, s_ref, acc_ref):
    j = pl.program_id(1)

    @pl.when(j == 0)
    def _init():
        m_ref[...] = jnp.full_like(m_ref, -jnp.inf)
        s_ref[...] = jnp.zeros_like(s_ref)
        acc_ref[...] = jnp.zeros_like(acc_ref)

    partials = []
    for c in range(_NCH):
        t = jnp.dot(trg_ref[0, c * _CH:(c + 1) * _CH, :], w_ref[...],
                    preferred_element_type=jnp.float32) + b_ref[...]
        sq = jnp.sum(t * t, axis=1, keepdims=True)
        norm = jnp.sqrt(sq)
        norm_ref[0, c] = norm
        mb = jnp.max(norm, axis=0, keepdims=True)
        p = jnp.exp(norm - mb)
        sb = jnp.sum(p, axis=0, keepdims=True)
        wsum = jnp.sum(t * p, axis=0, keepdims=True)
        partials.append((mb, sb, wsum))

    for mb, sb, wsum in partials:
        m_old = m_ref[...]
        m_new = jnp.maximum(m_old, mb)
        alpha = jnp.exp(m_old - m_new)
        beta = jnp.exp(mb - m_new[0:1, 0:1])
        s_ref[...] = alpha * s_ref[...] + sb * beta
        acc_ref[...] = alpha[0:1, 0:1] * acc_ref[...] + wsum * beta[0:1, 0:1]
        m_ref[...] = m_new

    @pl.when(j == _NL - 1)
    def _finish():
        summ_ref[0] = acc_ref[...] / s_ref[0:1, 0:1]


def kernel(trg, src, fc_w, fc_b):
    del src
    w_t = fc_w.T
    b2 = fc_b.reshape(1, _H)

    norm5, summ3 = pl.pallas_call(
        _decoder_kernel,
        grid=(_B, _NL),
        in_specs=[
            pl.BlockSpec((1, _LB, _K), lambda b, j: (b, j, 0)),
            pl.BlockSpec((_K, _H), lambda b, j: (0, 0)),
            pl.BlockSpec((1, _H), lambda b, j: (0, 0)),
        ],
        out_specs=[
            pl.BlockSpec((1, _NCH, _CH, 1), lambda b, j: (b * _NL + j, 0, 0, 0)),
            pl.BlockSpec((1, 1, _H), lambda b, j: (b, 0, 0)),
        ],
        out_shape=[
            jax.ShapeDtypeStruct((_B * _NL, _NCH, _CH, 1), jnp.float32),
            jax.ShapeDtypeStruct((_B, 1, _H), jnp.float32),
        ],
        scratch_shapes=[
            pltpu.VMEM((1, 128), jnp.float32),
            pltpu.VMEM((1, 128), jnp.float32),
            pltpu.VMEM((1, _H), jnp.float32),
        ],
        compiler_params=pltpu.CompilerParams(
            dimension_semantics=("parallel", "arbitrary"),
            vmem_limit_bytes=56 * 1024 * 1024,
        ),
    )(trg, w_t, b2)

    return summ3.reshape(_B, _H), norm5.reshape(_B, _L)


# speedup vs baseline: 1.2480x; 1.0519x over previous
import jax
import jax.numpy as jnp
from jax.experimental import pallas as pl
from jax.experimental.pallas import tpu as pltpu

_B, _L, _H = 32, 2048, 1024
_K = 3 * _H
_CH = 256
_NCH = 4
_LB = _CH * _NCH
_NL = _L // _LB


def _decoder_kernel(trg_ref, w_ref, b_ref, norm_ref, summ_ref,
                    m_ref, s_ref, acc_ref):
    j = pl.program_id(1)

    @pl.when(j == 0)
    def _init():
        m_ref[...] = jnp.full_like(m_ref, -jnp.inf)
        s_ref[...] = jnp.zeros_like(s_ref)
        acc_ref[...] = jnp.zeros_like(acc_ref)

    partials = []
    for c in range(_NCH):
        t = jax.lax.dot_general(
            trg_ref[0, c * _CH:(c + 1) * _CH, :], w_ref[...],
            (((1,), (1,)), ((), ())),
            preferred_element_type=jnp.float32) + b_ref[...]
        sq = jnp.sum(t * t, axis=1, keepdims=True)
        norm = jnp.sqrt(sq)
        norm_ref[0, c] = norm
        mb = jnp.max(norm, axis=0, keepdims=True)
        p = jnp.exp(norm - mb)
        sb = jnp.sum(p, axis=0, keepdims=True)
        wsum = jnp.sum(t * p, axis=0, keepdims=True)
        partials.append((mb, sb, wsum))

    for mb, sb, wsum in partials:
        m_old = m_ref[...]
        m_new = jnp.maximum(m_old, mb)
        alpha = jnp.exp(m_old - m_new)
        beta = jnp.exp(mb - m_new[0:1, 0:1])
        s_ref[...] = alpha * s_ref[...] + sb * beta
        acc_ref[...] = alpha[0:1, 0:1] * acc_ref[...] + wsum * beta[0:1, 0:1]
        m_ref[...] = m_new

    @pl.when(j == _NL - 1)
    def _finish():
        summ_ref[0] = acc_ref[...] / s_ref[0:1, 0:1]


def kernel(trg, src, fc_w, fc_b):
    del src
    b2 = fc_b.reshape(1, _H)

    norm5, summ3 = pl.pallas_call(
        _decoder_kernel,
        grid=(_B, _NL),
        in_specs=[
            pl.BlockSpec((1, _LB, _K), lambda b, j: (b, j, 0)),
            pl.BlockSpec((_H, _K), lambda b, j: (0, 0)),
            pl.BlockSpec((1, _H), lambda b, j: (0, 0)),
        ],
        out_specs=[
            pl.BlockSpec((1, _NCH, _CH, 1), lambda b, j: (b * _NL + j, 0, 0, 0)),
            pl.BlockSpec((1, 1, _H), lambda b, j: (b, 0, 0)),
        ],
        out_shape=[
            jax.ShapeDtypeStruct((_B * _NL, _NCH, _CH, 1), jnp.float32),
            jax.ShapeDtypeStruct((_B, 1, _H), jnp.float32),
        ],
        scratch_shapes=[
            pltpu.VMEM((1, 128), jnp.float32),
            pltpu.VMEM((1, 128), jnp.float32),
            pltpu.VMEM((1, _H), jnp.float32),
        ],
        compiler_params=pltpu.CompilerParams(
            dimension_semantics=("parallel", "arbitrary"),
            vmem_limit_bytes=56 * 1024 * 1024,
        ),
    )(trg, fc_w, b2)

    return summ3.reshape(_B, _H), norm5.reshape(_B, _L)
